# baseline (speedup 1.0000x reference)
import functools

import jax
import jax.numpy as jnp
import numpy as np
from jax.experimental import pallas as pl
from jax.experimental.pallas import tpu as pltpu

_CROSS_PCT = 0.25
_APPLY_PROB = 0.4
_N_ROWS = 20000
_D = 2000
_B = 1024
_BLK = 512

_APPLY1 = False
_APPLY2 = True
_IDX1 = 11085
_IDX2 = 18789
_MASK1_HEX = (
    "050822400090800a3a4480034687801004826ea09100b315041013e0e040042aa2"
    "0020008024c4c0d0209086004300fb54880428108d008a04024212724025148a82"
    "004001a090112822e00495220400091aa0c090000a1042220549084100a00503a5"
    "05e080542008881010806a32030c228906c2d5c6122008046208b12e0890212205"
    "8a0a120011300800a1004001652928900808800091c8030280004803186e00040e"
    "9208a011001814400048810148045256270200410268022a0d7504220058204007"
    "4ac0024c50c410208e0c541aa004809616846160081a0655ed9148850090840120"
    "0202090183c2103008030760a58419005e3260"
)
_MASK2_HEX = (
    "202100082008052292804215201881888b53f41060080308464441340083a090a2"
    "18800109a14870530881018c21c900207a80028084018008300a018002000e0002"
    "012000170685a0861849920186c005080e88c0528a846acd2d5007024271c53c88"
    "01000510140030b20801018081440c4104101320032400d03123587ca104048411"
    "147081004d4844848b442c11865e2620018224a0c622108d0805480b0002561080"
    "00404140e0500100041040d80c40030184284064aca1304390896a43005080a672"
    "04243389219818443103a80884000044140608aae128140880410882401060a000"
    "04524108015a07000011f100da102400600120"
)


def _unpack_mask(hex_bits):
    bits = np.unpackbits(np.frombuffer(bytes.fromhex(hex_bits), np.uint8))
    return bits[:_D].astype(bool)


_MASK1 = _unpack_mask(_MASK1_HEX)
_MASK2 = _unpack_mask(_MASK2_HEX)


_NC = 2
_NS = 16
_L = 16
_NW = _NC * _NS
_ROWS_PER_W = _B // _NW
_CHUNKS = _D // _L


def _make_blend():
    from jax import lax
    from jax.experimental.pallas import tpu_sc as plsc

    mesh = plsc.VectorSubcoreMesh(core_axis_name="c", subcore_axis_name="s")

    @functools.partial(
        pl.kernel,
        mesh=mesh,
        out_type=jax.ShapeDtypeStruct((_B, _D), jnp.float32),
        scratch_types=[
            pltpu.VMEM((_ROWS_PER_W, _D), jnp.float32),
            pltpu.VMEM((1, _D), jnp.float32),
            pltpu.VMEM((1, _D), jnp.float32),
        ],
    )
    def blend(x_hbm, row_hbm, m_hbm, out_hbm, x_v, row_v, m_v):
        c = lax.axis_index("c")
        s = lax.axis_index("s")
        wid = s * _NC + c
        base = wid * _ROWS_PER_W
        pltpu.sync_copy(x_hbm.at[pl.ds(base, _ROWS_PER_W)], x_v)
        pltpu.sync_copy(row_hbm, row_v)
        pltpu.sync_copy(m_hbm, m_v)

        def per_chunk(j, carry):
            off = j * _L
            pred = m_v[0, pl.ds(off, _L)] != 0.0
            rj = row_v[0, pl.ds(off, _L)]
            for r in range(_ROWS_PER_W):
                x_v[r, pl.ds(off, _L)] = jnp.where(
                    pred, rj, x_v[r, pl.ds(off, _L)])
            return carry

        lax.fori_loop(0, _CHUNKS, per_chunk, 0)
        pltpu.sync_copy(x_v, out_hbm.at[pl.ds(base, _ROWS_PER_W)])

    return blend


def kernel(x1, x2, cell_ids, X):
    outs = []
    for x, apply, idx, mask in (
        (x1, _APPLY1, _IDX1, _MASK1),
        (x2, _APPLY2, _IDX2, _MASK2),
    ):
        if apply:
            bank_row = jax.lax.slice(X, (idx, 0), (idx + 1, _D))
            m = jnp.asarray(mask.reshape(1, _D).astype(np.float32))
            outs.append(_make_blend()(x, bank_row, m))
        else:
            outs.append(x)
    return (outs[0], outs[1], cell_ids)


# speedup vs baseline: 1.6389x; 1.6389x over previous
import jax
import jax.numpy as jnp
import numpy as np
from jax.experimental import pallas as pl
from jax.experimental.pallas import tpu as pltpu

_CROSS_PCT = 0.25
_APPLY_PROB = 0.4
_N_ROWS = 20000
_D = 2000
_B = 1024
_BLK = 512

_APPLY1 = False
_APPLY2 = True
_IDX1 = 11085
_IDX2 = 18789
_MASK1_HEX = (
    "050822400090800a3a4480034687801004826ea09100b315041013e0e040042aa2"
    "0020008024c4c0d0209086004300fb54880428108d008a04024212724025148a82"
    "004001a090112822e00495220400091aa0c090000a1042220549084100a00503a5"
    "05e080542008881010806a32030c228906c2d5c6122008046208b12e0890212205"
    "8a0a120011300800a1004001652928900808800091c8030280004803186e00040e"
    "9208a011001814400048810148045256270200410268022a0d7504220058204007"
    "4ac0024c50c410208e0c541aa004809616846160081a0655ed9148850090840120"
    "0202090183c2103008030760a58419005e3260"
)
_MASK2_HEX = (
    "202100082008052292804215201881888b53f41060080308464441340083a090a2"
    "18800109a14870530881018c21c900207a80028084018008300a018002000e0002"
    "012000170685a0861849920186c005080e88c0528a846acd2d5007024271c53c88"
    "01000510140030b20801018081440c4104101320032400d03123587ca104048411"
    "147081004d4844848b442c11865e2620018224a0c622108d0805480b0002561080"
    "00404140e0500100041040d80c40030184284064aca1304390896a43005080a672"
    "04243389219818443103a80884000044140608aae128140880410882401060a000"
    "04524108015a07000011f100da102400600120"
)


def _unpack_mask(hex_bits):
    bits = np.unpackbits(np.frombuffer(bytes.fromhex(hex_bits), np.uint8))
    return bits[:_D].astype(bool)


_MASK1 = _unpack_mask(_MASK1_HEX)
_MASK2 = _unpack_mask(_MASK2_HEX)


def _make_blend():
    def body(x_ref, bank_ref, m_ref, o_ref):
        row = bank_ref[...]
        o_ref[...] = jnp.where(m_ref[...] != 0.0, row, x_ref[...])

    return pl.pallas_call(
        body,
        grid=(_B // _BLK,),
        in_specs=[
            pl.BlockSpec((_BLK, _D), lambda i: (i, 0)),
            pl.BlockSpec((1, _D), lambda i: (0, 0)),
            pl.BlockSpec((1, _D), lambda i: (0, 0)),
        ],
        out_specs=pl.BlockSpec((_BLK, _D), lambda i: (i, 0)),
        out_shape=jax.ShapeDtypeStruct((_B, _D), jnp.float32),
        input_output_aliases={0: 0},
    )


def kernel(x1, x2, cell_ids, X):
    outs = []
    for x, apply, idx, mask in (
        (x1, _APPLY1, _IDX1, _MASK1),
        (x2, _APPLY2, _IDX2, _MASK2),
    ):
        if apply:
            bank_row = jax.lax.slice(X, (idx, 0), (idx + 1, _D))
            m = jnp.asarray(mask.reshape(1, _D).astype(np.float32))
            outs.append(_make_blend()(x, bank_row, m))
        else:
            outs.append(x)
    return (outs[0], outs[1], cell_ids)


# speedup vs baseline: 1.6614x; 1.0138x over previous
import jax
import jax.numpy as jnp
import numpy as np
from jax.experimental import pallas as pl
from jax.experimental.pallas import tpu as pltpu

_CROSS_PCT = 0.25
_APPLY_PROB = 0.4
_N_ROWS = 20000
_D = 2000
_B = 1024
_BLK = 512

_APPLY1 = False
_APPLY2 = True
_IDX1 = 11085
_IDX2 = 18789
_MASK1_HEX = (
    "050822400090800a3a4480034687801004826ea09100b315041013e0e040042aa2"
    "0020008024c4c0d0209086004300fb54880428108d008a04024212724025148a82"
    "004001a090112822e00495220400091aa0c090000a1042220549084100a00503a5"
    "05e080542008881010806a32030c228906c2d5c6122008046208b12e0890212205"
    "8a0a120011300800a1004001652928900808800091c8030280004803186e00040e"
    "9208a011001814400048810148045256270200410268022a0d7504220058204007"
    "4ac0024c50c410208e0c541aa004809616846160081a0655ed9148850090840120"
    "0202090183c2103008030760a58419005e3260"
)
_MASK2_HEX = (
    "202100082008052292804215201881888b53f41060080308464441340083a090a2"
    "18800109a14870530881018c21c900207a80028084018008300a018002000e0002"
    "012000170685a0861849920186c005080e88c0528a846acd2d5007024271c53c88"
    "01000510140030b20801018081440c4104101320032400d03123587ca104048411"
    "147081004d4844848b442c11865e2620018224a0c622108d0805480b0002561080"
    "00404140e0500100041040d80c40030184284064aca1304390896a43005080a672"
    "04243389219818443103a80884000044140608aae128140880410882401060a000"
    "04524108015a07000011f100da102400600120"
)


def _unpack_mask(hex_bits):
    bits = np.unpackbits(np.frombuffer(bytes.fromhex(hex_bits), np.uint8))
    return bits[:_D].astype(bool)


_MASK1 = _unpack_mask(_MASK1_HEX)
_MASK2 = _unpack_mask(_MASK2_HEX)


def _make_blend():
    def body(x_ref, rm_ref, o_ref):
        row = rm_ref[0, :][None, :]
        m = rm_ref[1, :][None, :]
        o_ref[...] = jnp.where(m != 0.0, row, x_ref[...])

    return pl.pallas_call(
        body,
        grid=(_B // _BLK,),
        in_specs=[
            pl.BlockSpec((_BLK, _D), lambda i: (i, 0)),
            pl.BlockSpec((2, _D), lambda i: (0, 0)),
        ],
        out_specs=pl.BlockSpec((_BLK, _D), lambda i: (i, 0)),
        out_shape=jax.ShapeDtypeStruct((_B, _D), jnp.float32),
        input_output_aliases={0: 0},
    )


def kernel(x1, x2, cell_ids, X):
    outs = []
    for x, apply, idx, mask in (
        (x1, _APPLY1, _IDX1, _MASK1),
        (x2, _APPLY2, _IDX2, _MASK2),
    ):
        if apply:
            bank_row = jax.lax.slice(X, (idx, 0), (idx + 1, _D))
            m = jnp.asarray(mask.reshape(1, _D).astype(np.float32))
            row_and_mask = jnp.concatenate([bank_row, m], axis=0)
            outs.append(_make_blend()(x, row_and_mask))
        else:
            outs.append(x)
    return (outs[0], outs[1], cell_ids)
